# baseline (speedup 1.0000x reference)
import math

import jax
import jax.numpy as jnp
from jax.experimental import pallas as pl
from jax.experimental.pallas import tpu as pltpu

_L = 200
_B_BLK = 2048


def _counts_fast(pif):
    pb = pif.astype(jnp.bfloat16)
    cs = [jnp.sum((pb == jnp.bfloat16(k)).astype(jnp.bfloat16), axis=1,
                  keepdims=True).astype(jnp.float32)
          for k in range(10)]
    ben = [cs[0] + cs[1]] + cs[2:]
    return tuple(ben + cs)


def _counts_general(pif):
    nf = jnp.maximum(pif, 1.0)
    p10 = jnp.ones_like(nf)
    t = 10.0
    for _ in range(9):
        p10 = jnp.where(nf >= t, t, p10)
        t = t * 10.0
    fd = jnp.ones_like(nf)
    for k in range(2, 10):
        fd = fd + (nf >= float(k) * p10).astype(jnp.float32)
    last = pif - jnp.floor(pif / 10.0) * 10.0
    ben = [jnp.sum((fd == float(k)).astype(jnp.float32), axis=1, keepdims=True)
           for k in range(1, 10)]
    rnd = [jnp.sum((last == float(k)).astype(jnp.float32), axis=1, keepdims=True)
           for k in range(10)]
    return tuple(ben + rnd)


def _body(prices_ref, vol_ref, hold_ref, scal_ref, w_ref, bias_ref, out_ref):
    linv = 1.0 / _L
    dinv = 1.0 / (_L - 1)

    p = prices_ref[...]
    pif = jnp.floor(jnp.abs(p))
    counts = jax.lax.cond(jnp.max(pif) <= 9.0, _counts_fast, _counts_general, pif)

    feats = []
    for k in range(1, 10):
        ek = math.log10((k + 1.0) / float(k))
        feats.append(jnp.abs(counts[k - 1] * linv - ek))
    for k in range(10):
        feats.append(counts[9 + k] * linv)

    h = hold_ref[...]
    hsum = jnp.sum(h, axis=1, keepdims=True)
    hsq = jnp.sum(h * h, axis=1, keepdims=True)
    hm = hsum * linv
    hvar = jnp.maximum(hsq - hsum * hm, 0.0) * dinv
    feats.append(hm)
    feats.append(jnp.min(h, axis=1, keepdims=True))
    feats.append(jnp.max(h, axis=1, keepdims=True))
    feats.append(jnp.sqrt(hvar))

    v = vol_ref[...]
    vsum = jnp.sum(v, axis=1, keepdims=True)
    vsq = jnp.sum(v * v, axis=1, keepdims=True)
    vm = vsum * linv
    vvar = jnp.maximum(vsq - vsum * vm, 0.0) * dinv
    feats.append(scal_ref[:, 0:1])
    feats.append(scal_ref[:, 1:2])
    feats.append(scal_ref[:, 2:3])
    feats.append(vm)
    feats.append(jnp.sqrt(vvar))
    feats.append(vsum)

    acc = jnp.zeros((2048, 32), dtype=jnp.float32)
    for j, f in enumerate(feats):
        acc = acc + f * w_ref[j : j + 1, :]
    out_ref[...] = acc + bias_ref[...]


def kernel(prices, volumes, holding_times, unique_addresses, transaction_counts,
           contract_calls, W_benford, b_benford, W_round, b_round,
           W_turn, b_turn, W_act, b_act):
    B = prices.shape[0]
    scal = jnp.stack([unique_addresses, transaction_counts, contract_calls], axis=-1)
    wall = jax.scipy.linalg.block_diag(W_benford.T, W_round.T, W_turn.T, W_act.T)
    wall = jnp.pad(wall, ((0, 32 - wall.shape[0]), (0, 0)))
    bias = jnp.concatenate([b_benford, b_round, b_turn, b_act]).reshape(1, 32)

    grid = (B // _B_BLK,)
    return pl.pallas_call(
        _body,
        grid=grid,
        in_specs=[
            pl.BlockSpec((_B_BLK, _L), lambda i: (i, 0)),
            pl.BlockSpec((_B_BLK, _L), lambda i: (i, 0)),
            pl.BlockSpec((_B_BLK, _L), lambda i: (i, 0)),
            pl.BlockSpec((_B_BLK, 3), lambda i: (i, 0)),
            pl.BlockSpec((32, 32), lambda i: (0, 0)),
            pl.BlockSpec((1, 32), lambda i: (0, 0)),
        ],
        out_specs=pl.BlockSpec((_B_BLK, 32), lambda i: (i, 0)),
        out_shape=jax.ShapeDtypeStruct((B, 32), jnp.float32),
        compiler_params=pltpu.CompilerParams(
            dimension_semantics=("parallel",),
        ),
    )(prices, volumes, holding_times, scal, wall, bias)


# speedup vs baseline: 1.4103x; 1.4103x over previous
import math

import jax
import jax.numpy as jnp
from jax.experimental import pallas as pl
from jax.experimental.pallas import tpu as pltpu

_L = 200
_B_BLK = 2048


def _bincount(plane, k):
    m = plane == float(k)
    return jnp.sum(m, axis=1, keepdims=True, dtype=jnp.float32)


def _counts_fast(pif):
    cs = [_bincount(pif, k) for k in range(10)]
    ben = [cs[0] + cs[1]] + cs[2:]
    return tuple(ben + cs)


def _counts_general(pif):
    nf = jnp.maximum(pif, 1.0)
    p10 = jnp.ones_like(nf)
    t = 10.0
    for _ in range(9):
        p10 = jnp.where(nf >= t, t, p10)
        t = t * 10.0
    fd = jnp.ones_like(nf)
    for k in range(2, 10):
        fd = fd + (nf >= float(k) * p10).astype(jnp.float32)
    last = pif - jnp.floor(pif / 10.0) * 10.0
    ben = [_bincount(fd, k) for k in range(1, 10)]
    rnd = [_bincount(last, k) for k in range(10)]
    return tuple(ben + rnd)


def _tail(counts, hold_ref, vol_ref, scal_ref, wb_ref, wr_ref, wt_ref,
          wa_ref, bias_ref, out_ref):
    linv = 1.0 / _L
    dinv = 1.0 / (_L - 1)

    ben_acc = jnp.zeros((_B_BLK, 8), dtype=jnp.float32)
    for k in range(1, 10):
        ek = math.log10((k + 1.0) / float(k))
        f = jnp.abs(counts[k - 1] * linv - ek)
        ben_acc = ben_acc + f * wb_ref[k - 1 : k, :]
    rnd_acc = jnp.zeros((_B_BLK, 8), dtype=jnp.float32)
    for k in range(10):
        rnd_acc = rnd_acc + (counts[9 + k] * linv) * wr_ref[k : k + 1, :]

    h = hold_ref[...]
    hsum = jnp.sum(h, axis=1, keepdims=True)
    hsq = jnp.sum(h * h, axis=1, keepdims=True)
    hm = hsum * linv
    hvar = jnp.maximum(hsq - hsum * hm, 0.0) * dinv
    tfeat = (hm, jnp.min(h, axis=1, keepdims=True),
             jnp.max(h, axis=1, keepdims=True), jnp.sqrt(hvar))
    trn_acc = jnp.zeros((_B_BLK, 8), dtype=jnp.float32)
    for j, f in enumerate(tfeat):
        trn_acc = trn_acc + f * wt_ref[j : j + 1, :]

    v = vol_ref[...]
    vsum = jnp.sum(v, axis=1, keepdims=True)
    vsq = jnp.sum(v * v, axis=1, keepdims=True)
    vm = vsum * linv
    vvar = jnp.maximum(vsq - vsum * vm, 0.0) * dinv
    afeat = (scal_ref[:, 0:1], scal_ref[:, 1:2], scal_ref[:, 2:3],
             vm, jnp.sqrt(vvar), vsum)
    act_acc = jnp.zeros((_B_BLK, 8), dtype=jnp.float32)
    for j, f in enumerate(afeat):
        act_acc = act_acc + f * wa_ref[j : j + 1, :]

    out = jnp.concatenate([ben_acc, rnd_acc, trn_acc, act_acc], axis=1)
    out_ref[...] = out + bias_ref[...]


def _body_fast(prices_ref, vol_ref, hold_ref, scal_ref, wb_ref, wr_ref, wt_ref,
               wa_ref, bias_ref, out_ref, max_ref):
    pif = jnp.floor(jnp.abs(prices_ref[...]))
    max_ref[...] = jnp.full((8, 128), jnp.max(pif), dtype=jnp.float32)
    counts = _counts_fast(pif)
    _tail(counts, hold_ref, vol_ref, scal_ref, wb_ref, wr_ref, wt_ref,
          wa_ref, bias_ref, out_ref)


def _body_general(prices_ref, vol_ref, hold_ref, scal_ref, wb_ref, wr_ref,
                  wt_ref, wa_ref, bias_ref, out_ref):
    pif = jnp.floor(jnp.abs(prices_ref[...]))
    counts = _counts_general(pif)
    _tail(counts, hold_ref, vol_ref, scal_ref, wb_ref, wr_ref, wt_ref,
          wa_ref, bias_ref, out_ref)


def _pallas(body, B, n_out_extra):
    grid = (B // _B_BLK,)
    in_specs = [
        pl.BlockSpec((_B_BLK, _L), lambda i: (i, 0)),
        pl.BlockSpec((_B_BLK, _L), lambda i: (i, 0)),
        pl.BlockSpec((_B_BLK, _L), lambda i: (i, 0)),
        pl.BlockSpec((_B_BLK, 3), lambda i: (i, 0)),
        pl.BlockSpec((16, 8), lambda i: (0, 0)),
        pl.BlockSpec((16, 8), lambda i: (0, 0)),
        pl.BlockSpec((8, 8), lambda i: (0, 0)),
        pl.BlockSpec((8, 8), lambda i: (0, 0)),
        pl.BlockSpec((1, 32), lambda i: (0, 0)),
    ]
    out_specs = pl.BlockSpec((_B_BLK, 32), lambda i: (i, 0))
    out_shape = jax.ShapeDtypeStruct((B, 32), jnp.float32)
    if n_out_extra:
        out_specs = (out_specs, pl.BlockSpec((8, 128), lambda i: (i, 0)))
        out_shape = (out_shape,
                     jax.ShapeDtypeStruct((8 * (B // _B_BLK), 128), jnp.float32))
    return pl.pallas_call(
        body,
        grid=grid,
        in_specs=in_specs,
        out_specs=out_specs,
        out_shape=out_shape,
        compiler_params=pltpu.CompilerParams(
            dimension_semantics=("parallel",),
        ),
    )


def kernel(prices, volumes, holding_times, unique_addresses, transaction_counts,
           contract_calls, W_benford, b_benford, W_round, b_round,
           W_turn, b_turn, W_act, b_act):
    B = prices.shape[0]
    scal = jnp.stack([unique_addresses, transaction_counts, contract_calls], axis=-1)
    wb = jnp.pad(W_benford.T, ((0, 7), (0, 0)))
    wr = jnp.pad(W_round.T, ((0, 6), (0, 0)))
    wt = jnp.pad(W_turn.T, ((0, 4), (0, 0)))
    wa = jnp.pad(W_act.T, ((0, 2), (0, 0)))
    bias = jnp.concatenate([b_benford, b_round, b_turn, b_act]).reshape(1, 32)

    args = (prices, volumes, holding_times, scal, wb, wr, wt, wa, bias)
    fast_out, blockmax = _pallas(_body_fast, B, 1)(*args)
    has_big = jnp.max(blockmax) > 9.0
    return jax.lax.cond(
        has_big,
        lambda a: _pallas(_body_general, B, 0)(*a),
        lambda a: fast_out,
        args)


# speedup vs baseline: 1.4454x; 1.0249x over previous
import math

import jax
import jax.numpy as jnp
from jax.experimental import pallas as pl
from jax.experimental.pallas import tpu as pltpu

_L = 200
_B_BLK = 2048


def _bincount(plane, k):
    m = plane == float(k)
    return jnp.sum(m, axis=1, keepdims=True, dtype=jnp.float32)


def _counts_fast(pif):
    cs = []
    for g in range(3):
        b = 3 * g
        y = jnp.where(pif == float(b), 1.0,
                      jnp.where(pif == float(b + 1), 256.0,
                                jnp.where(pif == float(b + 2), 65536.0, 0.0)))
        s = jnp.sum(y, axis=1, keepdims=True)
        c2 = jnp.floor(s * (1.0 / 65536.0))
        rem = s - c2 * 65536.0
        c1 = jnp.floor(rem * (1.0 / 256.0))
        c0 = rem - c1 * 256.0
        cs.extend([c0, c1, c2])
    cs.append(_bincount(pif, 9))
    ben = [cs[0] + cs[1]] + cs[2:]
    return tuple(ben + cs)


def _counts_general(pif):
    nf = jnp.maximum(pif, 1.0)
    p10 = jnp.ones_like(nf)
    t = 10.0
    for _ in range(9):
        p10 = jnp.where(nf >= t, t, p10)
        t = t * 10.0
    fd = jnp.ones_like(nf)
    for k in range(2, 10):
        fd = fd + (nf >= float(k) * p10).astype(jnp.float32)
    last = pif - jnp.floor(pif / 10.0) * 10.0
    ben = [_bincount(fd, k) for k in range(1, 10)]
    rnd = [_bincount(last, k) for k in range(10)]
    return tuple(ben + rnd)


def _tail(counts, hold_ref, vol_ref, scal_ref, wb_ref, wr_ref, wt_ref,
          wa_ref, bias_ref, out_ref):
    linv = 1.0 / _L
    dinv = 1.0 / (_L - 1)

    ben_acc = jnp.zeros((_B_BLK, 8), dtype=jnp.float32)
    for k in range(1, 10):
        ek = math.log10((k + 1.0) / float(k))
        f = jnp.abs(counts[k - 1] * linv - ek)
        ben_acc = ben_acc + f * wb_ref[k - 1 : k, :]
    rnd_acc = jnp.zeros((_B_BLK, 8), dtype=jnp.float32)
    for k in range(10):
        rnd_acc = rnd_acc + (counts[9 + k] * linv) * wr_ref[k : k + 1, :]

    h = hold_ref[...]
    hsum = jnp.sum(h, axis=1, keepdims=True)
    hsq = jnp.sum(h * h, axis=1, keepdims=True)
    hm = hsum * linv
    hvar = jnp.maximum(hsq - hsum * hm, 0.0) * dinv
    tfeat = (hm, jnp.min(h, axis=1, keepdims=True),
             jnp.max(h, axis=1, keepdims=True), jnp.sqrt(hvar))
    trn_acc = jnp.zeros((_B_BLK, 8), dtype=jnp.float32)
    for j, f in enumerate(tfeat):
        trn_acc = trn_acc + f * wt_ref[j : j + 1, :]

    v = vol_ref[...]
    vsum = jnp.sum(v, axis=1, keepdims=True)
    vsq = jnp.sum(v * v, axis=1, keepdims=True)
    vm = vsum * linv
    vvar = jnp.maximum(vsq - vsum * vm, 0.0) * dinv
    afeat = (scal_ref[:, 0:1], scal_ref[:, 1:2], scal_ref[:, 2:3],
             vm, jnp.sqrt(vvar), vsum)
    act_acc = jnp.zeros((_B_BLK, 8), dtype=jnp.float32)
    for j, f in enumerate(afeat):
        act_acc = act_acc + f * wa_ref[j : j + 1, :]

    out = jnp.concatenate([ben_acc, rnd_acc, trn_acc, act_acc], axis=1)
    out_ref[...] = out + bias_ref[...]


def _body_fast(prices_ref, vol_ref, hold_ref, scal_ref, wb_ref, wr_ref, wt_ref,
               wa_ref, bias_ref, out_ref, max_ref):
    pif = jnp.floor(jnp.abs(prices_ref[...]))
    max_ref[...] = jnp.full((8, 128), jnp.max(pif), dtype=jnp.float32)
    counts = _counts_fast(pif)
    _tail(counts, hold_ref, vol_ref, scal_ref, wb_ref, wr_ref, wt_ref,
          wa_ref, bias_ref, out_ref)


def _body_general(prices_ref, vol_ref, hold_ref, scal_ref, wb_ref, wr_ref,
                  wt_ref, wa_ref, bias_ref, out_ref):
    pif = jnp.floor(jnp.abs(prices_ref[...]))
    counts = _counts_general(pif)
    _tail(counts, hold_ref, vol_ref, scal_ref, wb_ref, wr_ref, wt_ref,
          wa_ref, bias_ref, out_ref)


def _pallas(body, B, n_out_extra):
    grid = (B // _B_BLK,)
    in_specs = [
        pl.BlockSpec((_B_BLK, _L), lambda i: (i, 0)),
        pl.BlockSpec((_B_BLK, _L), lambda i: (i, 0)),
        pl.BlockSpec((_B_BLK, _L), lambda i: (i, 0)),
        pl.BlockSpec((_B_BLK, 3), lambda i: (i, 0)),
        pl.BlockSpec((16, 8), lambda i: (0, 0)),
        pl.BlockSpec((16, 8), lambda i: (0, 0)),
        pl.BlockSpec((8, 8), lambda i: (0, 0)),
        pl.BlockSpec((8, 8), lambda i: (0, 0)),
        pl.BlockSpec((1, 32), lambda i: (0, 0)),
    ]
    out_specs = pl.BlockSpec((_B_BLK, 32), lambda i: (i, 0))
    out_shape = jax.ShapeDtypeStruct((B, 32), jnp.float32)
    if n_out_extra:
        out_specs = (out_specs, pl.BlockSpec((8, 128), lambda i: (i, 0)))
        out_shape = (out_shape,
                     jax.ShapeDtypeStruct((8 * (B // _B_BLK), 128), jnp.float32))
    return pl.pallas_call(
        body,
        grid=grid,
        in_specs=in_specs,
        out_specs=out_specs,
        out_shape=out_shape,
        compiler_params=pltpu.CompilerParams(
            dimension_semantics=("parallel",),
        ),
    )


def kernel(prices, volumes, holding_times, unique_addresses, transaction_counts,
           contract_calls, W_benford, b_benford, W_round, b_round,
           W_turn, b_turn, W_act, b_act):
    B = prices.shape[0]
    scal = jnp.stack([unique_addresses, transaction_counts, contract_calls], axis=-1)
    wb = jnp.pad(W_benford.T, ((0, 7), (0, 0)))
    wr = jnp.pad(W_round.T, ((0, 6), (0, 0)))
    wt = jnp.pad(W_turn.T, ((0, 4), (0, 0)))
    wa = jnp.pad(W_act.T, ((0, 2), (0, 0)))
    bias = jnp.concatenate([b_benford, b_round, b_turn, b_act]).reshape(1, 32)

    args = (prices, volumes, holding_times, scal, wb, wr, wt, wa, bias)
    fast_out, blockmax = _pallas(_body_fast, B, 1)(*args)
    has_big = jnp.max(blockmax) > 9.0
    return jax.lax.cond(
        has_big,
        lambda a: _pallas(_body_general, B, 0)(*a),
        lambda a: fast_out,
        args)
